# baseline (speedup 1.0000x reference)
import functools

import jax
import jax.numpy as jnp
from jax import lax
from jax.experimental import pallas as pl
from jax.experimental.pallas import tpu as pltpu
from jax.experimental.pallas import tpu_sc as plsc

_NW = 32
_B = 16384 * 32
_D = 128
_V = 32
_G = 128
_PER_W = _B // _NW
_NGRP = _PER_W // _G
_L = 16

_mesh = plsc.VectorSubcoreMesh(core_axis_name="c", subcore_axis_name="s")


@functools.partial(
    pl.kernel,
    mesh=_mesh,
    out_type=jax.ShapeDtypeStruct((_B * _D,), jnp.float32),
    compiler_params=pltpu.CompilerParams(needs_layout_passes=False),
    scratch_types=[
        pltpu.VMEM((_PER_W,), jnp.int32),
        pltpu.VMEM((_V * _D,), jnp.float32),
        pltpu.VMEM((_G * _D,), jnp.float32),
        pltpu.VMEM((_G * _D,), jnp.float32),
        pltpu.SemaphoreType.DMA,
        pltpu.SemaphoreType.DMA,
    ],
)
def _gather_all(idx_hbm, table_hbm, out_hbm, idx_v, tab_v, b0, b1, w0, w1):
    wid = lax.axis_index("s") * 2 + lax.axis_index("c")
    base = wid * _PER_W
    pltpu.sync_copy(idx_hbm.at[wid], idx_v)
    pltpu.sync_copy(table_hbm, tab_v)

    def w_start(buf, sem, g):
        pltpu.async_copy(buf, out_hbm.at[pl.ds((base + g * _G) * _D, _G * _D)],
                         sem)

    def w_wait(buf, sem):
        pltpu.make_async_copy(buf, out_hbm.at[pl.ds(base * _D, _G * _D)],
                              sem).wait()

    def build(g, buf):
        @plsc.parallel_loop(0, _G // _L, unroll=2)
        def sgbody(r16):
            src16 = idx_v[pl.ds(g * _G + r16 * _L, _L)] * _D
            dst16 = r16 * _L * _D
            for l in range(_L):
                src = src16[l]
                dst = dst16 + l * _D
                for c0 in range(_D // _L):
                    buf[pl.ds(dst + c0 * _L, _L)] = (
                        tab_v[pl.ds(src + c0 * _L, _L)])

    build(0, b0)
    w_start(b0, w0, 0)
    build(1, b1)
    w_start(b1, w1, 1)

    def body(t, carry):
        g = 2 * t
        w_wait(b0, w0)
        build(g, b0)
        w_start(b0, w0, g)
        w_wait(b1, w1)
        build(g + 1, b1)
        w_start(b1, w1, g + 1)
        return carry

    lax.fori_loop(1, _NGRP // 2, body, 0)
    w_wait(b0, w0)
    w_wait(b1, w1)


def kernel(x, weight):
    xr = x.reshape(_NW, _PER_W)
    out = _gather_all(xr, weight.reshape(_V * _D))
    return out.reshape(16384, 32, _D)


# speedup vs baseline: 1.4964x; 1.4964x over previous
import functools

import jax
import jax.numpy as jnp
from jax import lax
from jax.experimental import pallas as pl
from jax.experimental.pallas import tpu as pltpu
from jax.experimental.pallas import tpu_sc as plsc

_NW = 32
_B = 16384 * 32
_D = 128
_V = 32
_G = 128
_PER_W = _B // _NW
_NGRP = _PER_W // _G

_mesh = plsc.VectorSubcoreMesh(core_axis_name="c", subcore_axis_name="s")


@functools.partial(
    pl.kernel,
    mesh=_mesh,
    out_type=jax.ShapeDtypeStruct((_B, _D), jnp.float32),
    compiler_params=pltpu.CompilerParams(needs_layout_passes=False),
    scratch_types=[
        pltpu.VMEM((_NGRP, _G), jnp.int32),
        pltpu.VMEM((_G, _D), jnp.float32),
        pltpu.VMEM((_G, _D), jnp.float32),
        pltpu.VMEM_SHARED((_V, _D), jnp.float32),
        pltpu.SemaphoreType.DMA,
        pltpu.SemaphoreType.DMA,
        pltpu.SemaphoreType.DMA,
        pltpu.SemaphoreType.DMA,
    ],
)
def _gather_all(idx_hbm, table_hbm, out_hbm, idx_v, b0, b1, tab_sh,
                g0, g1, w0, w1):
    sid = lax.axis_index("s")
    wid = sid * 2 + lax.axis_index("c")
    base = wid * _PER_W

    @pl.when(sid == 0)
    def _():
        pltpu.sync_copy(table_hbm, tab_sh)

    pltpu.sync_copy(idx_hbm.at[wid], idx_v)
    plsc.subcore_barrier()

    bufs = (b0, b1)
    gsems = (g0, g1)
    wsems = (w0, w1)

    def g_start(b, g):
        pltpu.async_copy(tab_sh.at[idx_v.at[g]], bufs[b], gsems[b])

    def g_wait(b):
        pltpu.make_async_copy(tab_sh.at[idx_v.at[0]], bufs[b], gsems[b]).wait()

    def w_start(b, g):
        pltpu.async_copy(bufs[b], out_hbm.at[pl.ds(base + g * _G, _G)],
                         wsems[b])

    def w_wait(b):
        pltpu.make_async_copy(bufs[b], out_hbm.at[pl.ds(base, _G)],
                              wsems[b]).wait()

    g_start(0, 0)
    g_start(1, 1)
    g_wait(0)
    w_start(0, 0)
    g_wait(1)
    w_start(1, 1)

    def body(t, carry):
        g = 2 * t
        w_wait(0)
        g_start(0, g)
        w_wait(1)
        g_start(1, g + 1)
        g_wait(0)
        w_start(0, g)
        g_wait(1)
        w_start(1, g + 1)
        return carry

    lax.fori_loop(1, _NGRP // 2, body, 0)
    w_wait(0)
    w_wait(1)


def kernel(x, weight):
    xr = x.reshape(_NW, _NGRP, _G)
    out = _gather_all(xr, weight)
    return out.reshape(16384, 32, _D)
